# baseline (speedup 1.0000x reference)
import functools

import jax
import jax.numpy as jnp
from jax import lax
from jax.experimental import pallas as pl
from jax.experimental.pallas import tpu as pltpu
from jax.experimental.pallas import tpu_sc as plsc

NC = 2
NS = 16
NW = NC * NS
L = 16

F = 26
D = 32
CHUNK = 64
M = CHUNK * F
NSTREAM = M // 128


def _body(inputs_hbm, table_hbm, offt_hbm, out_hbm,
          in_v, off_v, idx_v, buf_v, out_v, sem):
    wid = lax.axis_index("s") * NC + lax.axis_index("c")
    n_chunks = out_hbm.shape[0] // (NW * CHUNK)

    pltpu.sync_copy(offt_hbm, off_v)

    def chunk_body(c, carry):
        base = (wid * n_chunks + c) * M
        pltpu.sync_copy(inputs_hbm.at[pl.ds(base, M)], in_v)

        def idx_body(j, carry2):
            for l in range(128 // L):
                s = j * 128 + l * L
                idx_v[j, pl.ds(l * L, L)] = (
                    in_v[pl.ds(s, L)] + off_v[pl.ds(s, L)])
            return carry2
        lax.fori_loop(0, NSTREAM, idx_body, 0, unroll=False)

        descs = [
            pltpu.async_copy(table_hbm.at[idx_v.at[j]],
                             buf_v.at[pl.ds(j * 128, 128)], sem)
            for j in range(NSTREAM)
        ]
        for d in descs:
            d.wait()

        def sum_body(r, carry2):
            g = r * F
            acc0 = buf_v[g, pl.ds(0, L)]
            acc1 = buf_v[g, pl.ds(L, L)]
            for f in range(1, F):
                acc0 = acc0 + buf_v[g + f, pl.ds(0, L)]
                acc1 = acc1 + buf_v[g + f, pl.ds(L, L)]
            out_v[r, pl.ds(0, L)] = acc0
            out_v[r, pl.ds(L, L)] = acc1
            return carry2
        lax.fori_loop(0, CHUNK, sum_body, 0, unroll=False)

        pltpu.sync_copy(out_v, out_hbm.at[pl.ds((wid * n_chunks + c) * CHUNK,
                                                CHUNK)])
        return carry

    lax.fori_loop(0, n_chunks, chunk_body, 0, unroll=False)


def kernel(inputs, table, offset):
    B = inputs.shape[0]
    inputs_flat = inputs.reshape(B * F)
    off_tiled = jnp.tile(offset, CHUNK)

    k = pl.kernel(
        _body,
        out_type=jax.ShapeDtypeStruct((B, D), jnp.float32),
        mesh=plsc.VectorSubcoreMesh(core_axis_name="c", subcore_axis_name="s"),
        scratch_types=[
            pltpu.VMEM((M,), jnp.int32),
            pltpu.VMEM((M,), jnp.int32),
            pltpu.VMEM((NSTREAM, 128), jnp.int32),
            pltpu.VMEM((M, D), jnp.float32),
            pltpu.VMEM((CHUNK, D), jnp.float32),
            pltpu.SemaphoreType.DMA,
        ],
        compiler_params=pltpu.CompilerParams(use_tc_tiling_on_sc=False),
    )
    return k(inputs_flat, table, off_tiled)
